# baseline (speedup 1.0000x reference)
import functools

import jax
import jax.numpy as jnp
from jax import lax
from jax.experimental import pallas as pl
from jax.experimental.pallas import tpu as pltpu
from jax.experimental.pallas import tpu_sc as plsc

NUM_E = 1024
DIM = 64
BLK_M = 1024

_SC_INFO = plsc.get_sparse_core_info()
_NC = _SC_INFO.num_cores
_NW = _SC_INFO.num_cores * _SC_INFO.num_subcores
_CHUNK = 96


def _dist_block(x_ref, emb_ref, sx_ref, se_ref, dist_ref, idx_ref):
    x = x_ref[...]
    sx = sx_ref[...]
    se = se_ref[...]
    mm = jax.lax.dot_general(x, emb_ref[...], (((1,), (1,)), ((), ())),
                             preferred_element_type=jnp.float32)
    d = sx - 2.0 * mm + se
    dist_ref[...] = d
    mn = jnp.min(d, axis=1, keepdims=True)
    iota = jax.lax.broadcasted_iota(jnp.int32, (x.shape[0], NUM_E), 1)
    cand = jnp.where(d == mn, iota, NUM_E)
    idx_ref[...] = jnp.min(cand, axis=1)[:, None]


def _enc_block(idx_ref, enc_ref, perp_ref, hist_ref, *, total_rows):
    idxv = idx_ref[...]
    iota = jax.lax.broadcasted_iota(jnp.int32, (idxv.shape[0], NUM_E), 1)
    onehot = (iota == idxv).astype(jnp.float32)
    enc_ref[...] = onehot

    i = pl.program_id(0)

    @pl.when(i == 0)
    def _init():
        hist_ref[...] = jnp.zeros_like(hist_ref)

    hist_ref[...] += jnp.sum(onehot, axis=0, keepdims=True)

    @pl.when(i == pl.num_programs(0) - 1)
    def _fin():
        avg = hist_ref[...] / float(total_rows)
        p = jnp.exp(-jnp.sum(avg * jnp.log(avg + 1e-10)))
        perp_ref[...] = p[None, None]


def _sc_gather_kernel(b_per_w):
    n_chunks = b_per_w // _CHUNK

    @functools.partial(
        pl.kernel,
        mesh=plsc.VectorSubcoreMesh(core_axis_name="c", subcore_axis_name="s"),
        out_type=jax.ShapeDtypeStruct((b_per_w * _NW, DIM), jnp.float32),
        scratch_types=[
            pltpu.VMEM((b_per_w,), jnp.int32),
            pltpu.VMEM((b_per_w, DIM), jnp.float32),
            pltpu.SemaphoreType.DMA,
        ],
        compiler_params=pltpu.CompilerParams(use_tc_tiling_on_sc=False),
    )
    def gather(emb_hbm, idx_hbm, out_hbm, idx_v, rows_v, sem):
        wid = lax.axis_index("s") * _NC + lax.axis_index("c")
        base = wid * b_per_w
        pltpu.sync_copy(idx_hbm.at[pl.ds(base, b_per_w)], idx_v)
        copies = []
        for j in range(n_chunks):
            lo = j * _CHUNK
            copies.append(pltpu.async_copy(
                emb_hbm.at[idx_v.at[pl.ds(lo, _CHUNK)]],
                rows_v.at[pl.ds(lo, _CHUNK), :], sem))
        for c in copies:
            c.wait()
        pltpu.sync_copy(rows_v, out_hbm.at[pl.ds(base, b_per_w)])

    return gather


@jax.jit
def kernel(inputs, emb):
    m = inputs.shape[0] * inputs.shape[1]
    flat = inputs.reshape(m, DIM)
    sx = jnp.sum(flat ** 2, axis=1, keepdims=True)
    se = jnp.sum(emb ** 2, axis=1)[None, :]
    n_blocks = m // BLK_M
    dist, idx = pl.pallas_call(
        _dist_block,
        grid=(n_blocks,),
        in_specs=[
            pl.BlockSpec((BLK_M, DIM), lambda i: (i, 0)),
            pl.BlockSpec((NUM_E, DIM), lambda i: (0, 0)),
            pl.BlockSpec((BLK_M, 1), lambda i: (i, 0)),
            pl.BlockSpec((1, NUM_E), lambda i: (0, 0)),
        ],
        out_specs=[
            pl.BlockSpec((BLK_M, NUM_E), lambda i: (i, 0)),
            pl.BlockSpec((BLK_M, 1), lambda i: (i, 0)),
        ],
        out_shape=[
            jax.ShapeDtypeStruct((m, NUM_E), jnp.float32),
            jax.ShapeDtypeStruct((m, 1), jnp.int32),
        ],
    )(flat, emb, sx, se)
    quant = _sc_gather_kernel(m // _NW)(emb, idx.reshape(m))
    enc, perp = pl.pallas_call(
        functools.partial(_enc_block, total_rows=m),
        grid=(n_blocks,),
        in_specs=[pl.BlockSpec((BLK_M, 1), lambda i: (i, 0))],
        out_specs=[
            pl.BlockSpec((BLK_M, NUM_E), lambda i: (i, 0)),
            pl.BlockSpec((1, 1), lambda i: (0, 0)),
        ],
        out_shape=[
            jax.ShapeDtypeStruct((m, NUM_E), jnp.float32),
            jax.ShapeDtypeStruct((1, 1), jnp.float32),
        ],
        scratch_shapes=[pltpu.VMEM((1, NUM_E), jnp.float32)],
    )(idx)
    quantized = quant.reshape(inputs.shape)
    enc_idx = idx.reshape(inputs.shape[:-1])
    return (quantized, perp.reshape(()), enc, enc_idx, dist)


# speedup vs baseline: 1.2916x; 1.2916x over previous
import functools

import jax
import jax.numpy as jnp
from jax.experimental import pallas as pl
from jax.experimental.pallas import tpu as pltpu

NUM_E = 1024
DIM = 64
BLK_M = 2048


def _vq_block(x_ref, emb_ref, sx_ref, se_ref, dist_ref, enc_ref, idx_ref,
              quant_ref, perp_ref, hist_ref, *, total_rows):
    x = x_ref[...]
    e = emb_ref[...]
    sx = sx_ref[...]
    se = se_ref[...]
    mm = jax.lax.dot_general(x, e, (((1,), (1,)), ((), ())),
                             preferred_element_type=jnp.float32)
    d = sx - 2.0 * mm + se
    dist_ref[...] = d
    mn = jnp.min(d, axis=1, keepdims=True)
    iota = jax.lax.broadcasted_iota(jnp.int32, (x.shape[0], NUM_E), 1)
    cand = jnp.where(d == mn, iota, NUM_E)
    idx = jnp.min(cand, axis=1)
    idx_ref[...] = idx[:, None]
    onehot = (cand == idx[:, None]).astype(jnp.float32)
    enc_ref[...] = onehot
    q = jax.lax.dot_general(onehot, e, (((1,), (0,)), ((), ())),
                            preferred_element_type=jnp.float32)
    quant_ref[...] = x + (q - x)

    i = pl.program_id(0)

    @pl.when(i == 0)
    def _init():
        hist_ref[...] = jnp.zeros_like(hist_ref)

    hist_ref[...] += jnp.sum(onehot, axis=0, keepdims=True)

    @pl.when(i == pl.num_programs(0) - 1)
    def _fin():
        avg = hist_ref[...] / float(total_rows)
        p = jnp.exp(-jnp.sum(avg * jnp.log(avg + 1e-10)))
        perp_ref[...] = p[None, None]


@jax.jit
def kernel(inputs, emb):
    m = inputs.shape[0] * inputs.shape[1]
    flat = inputs.reshape(m, DIM)
    sx = jnp.sum(flat ** 2, axis=1, keepdims=True)
    se = jnp.sum(emb ** 2, axis=1)[None, :]
    n_blocks = m // BLK_M
    dist, enc, idx, quant, perp = pl.pallas_call(
        functools.partial(_vq_block, total_rows=m),
        grid=(n_blocks,),
        in_specs=[
            pl.BlockSpec((BLK_M, DIM), lambda i: (i, 0)),
            pl.BlockSpec((NUM_E, DIM), lambda i: (0, 0)),
            pl.BlockSpec((BLK_M, 1), lambda i: (i, 0)),
            pl.BlockSpec((1, NUM_E), lambda i: (0, 0)),
        ],
        out_specs=[
            pl.BlockSpec((BLK_M, NUM_E), lambda i: (i, 0)),
            pl.BlockSpec((BLK_M, NUM_E), lambda i: (i, 0)),
            pl.BlockSpec((BLK_M, 1), lambda i: (i, 0)),
            pl.BlockSpec((BLK_M, DIM), lambda i: (i, 0)),
            pl.BlockSpec((1, 1), lambda i: (0, 0)),
        ],
        out_shape=[
            jax.ShapeDtypeStruct((m, NUM_E), jnp.float32),
            jax.ShapeDtypeStruct((m, NUM_E), jnp.float32),
            jax.ShapeDtypeStruct((m, 1), jnp.int32),
            jax.ShapeDtypeStruct((m, DIM), jnp.float32),
            jax.ShapeDtypeStruct((1, 1), jnp.float32),
        ],
        scratch_shapes=[pltpu.VMEM((1, NUM_E), jnp.float32)],
    )(flat, emb, sx, se)
    quantized = quant.reshape(inputs.shape)
    enc_idx = idx.reshape(inputs.shape[:-1])
    return (quantized, perp.reshape(()), enc, enc_idx, dist)
